# baseline (speedup 1.0000x reference)
import functools

import jax
import jax.numpy as jnp
from jax import lax
from jax.experimental import pallas as pl
from jax.experimental.pallas import tpu as pltpu
from jax.experimental.pallas import tpu_sc as plsc

N = 10000
E = 320000
DF = 128
H = 16
G = 64
NCLS = 2

NC = 2
NS = 16
NW = NC * NS
LW = 128
EP = 327680
ROWS = EP // LW
RPW = ROWS // NW
CR = 8
CHUNKS = RPW // CR
NPAD = 10112
RPS = NPAD // NS
DPAD = 10240
DPS = DPAD // NS

_sc_mesh = plsc.VectorSubcoreMesh(
    core_axis_name="c", subcore_axis_name="s", num_cores=NC, num_subcores=NS)

_sc_params = pltpu.CompilerParams(use_tc_tiling_on_sc=False)


def _deg_body(dstr, out, didx, ones_v, zbuf, dacc):
    c = lax.axis_index("c")
    s = lax.axis_index("s")
    w = c * NS + s

    def _zero(j, carry):
        zbuf[pl.ds(j * 16, 16)] = jnp.zeros((16,), jnp.float32)
        return carry

    lax.fori_loop(0, DPS // 16, _zero, 0)

    def _one(j, carry):
        ones_v[pl.ds(j * 16, 16)] = jnp.ones((16,), jnp.float32)
        return carry

    lax.fori_loop(0, LW // 16, _one, 0)

    pltpu.sync_copy(zbuf, dacc.at[pl.ds(s * DPS, DPS)])
    plsc.subcore_barrier()

    def _chunk(ci, carry):
        base = w * RPW + ci * CR
        pltpu.sync_copy(dstr.at[pl.ds(base, CR)], didx)
        for j in range(CR):
            pltpu.sync_copy(ones_v, dacc.at[didx.at[j]], add=True)
        return carry

    lax.fori_loop(0, CHUNKS, _chunk, 0)
    plsc.subcore_barrier()
    pltpu.sync_copy(dacc.at[pl.ds(s * DPS, DPS)], out.at[c, pl.ds(s * DPS, DPS)])


_deg_kernel = pl.kernel(
    _deg_body,
    out_type=jax.ShapeDtypeStruct((NC, DPAD), jnp.float32),
    mesh=_sc_mesh,
    scratch_types=[
        pltpu.VMEM((CR, LW), jnp.int32),
        pltpu.VMEM((LW,), jnp.float32),
        pltpu.VMEM((DPS,), jnp.float32),
        pltpu.VMEM_SHARED((DPAD,), jnp.float32),
    ],
    compiler_params=_sc_params,
)


def _conv_body(g, srcr, dstr, out, sidx, didx, rows, zbuf, acc, sems):
    c = lax.axis_index("c")
    s = lax.axis_index("s")
    w = (1 - c) * NS + s

    def _zero(j, carry):
        zbuf[j, :] = jnp.zeros((16,), jnp.float32)
        return carry

    lax.fori_loop(0, RPS, _zero, 0)
    pltpu.sync_copy(zbuf, acc.at[pl.ds(s * RPS, RPS)])
    plsc.subcore_barrier()

    def _chunk(ci, carry):
        base = w * RPW + ci * CR
        pltpu.sync_copy(srcr.at[pl.ds(base, CR)], sidx)
        pltpu.sync_copy(dstr.at[pl.ds(base, CR)], didx)
        copies = [
            pltpu.async_copy(g.at[sidx.at[j]],
                             rows.at[pl.ds(j * LW, LW)], sems.at[j])
            for j in range(CR)
        ]
        for j in range(CR):
            copies[j].wait()
            pltpu.sync_copy(rows.at[pl.ds(j * LW, LW)],
                            acc.at[didx.at[j]], add=True)
        return carry

    lax.fori_loop(0, CHUNKS, _chunk, 0)
    plsc.subcore_barrier()
    pltpu.sync_copy(acc.at[pl.ds(s * RPS, RPS)], out.at[c, pl.ds(s * RPS, RPS)])


_conv_kernel = pl.kernel(
    _conv_body,
    out_type=jax.ShapeDtypeStruct((NC, NPAD, H), jnp.float32),
    mesh=_sc_mesh,
    scratch_types=[
        pltpu.VMEM((CR, LW), jnp.int32),
        pltpu.VMEM((CR, LW), jnp.int32),
        pltpu.VMEM((CR * LW, H), jnp.float32),
        pltpu.VMEM((RPS, H), jnp.float32),
        pltpu.VMEM_SHARED((NPAD, H), jnp.float32),
        pltpu.SemaphoreType.DMA((CR,)),
    ],
    compiler_params=_sc_params,
)


def _tc1_body(x_ref, w1_ref, degt_ref, g_ref, dinv_ref):
    deg = degt_ref[:, 0:1] + degt_ref[:, 1:2] + 1.0
    dinv = lax.rsqrt(deg)
    dinv_ref[...] = dinv
    h = jnp.dot(x_ref[...], w1_ref[...], preferred_element_type=jnp.float32)
    g_ref[...] = h * dinv


_tc1 = pl.pallas_call(
    _tc1_body,
    out_shape=(
        jax.ShapeDtypeStruct((N, H), jnp.float32),
        jax.ShapeDtypeStruct((N, 1), jnp.float32),
    ),
)


def _tc2_body(s1_ref, g1_ref, dinv_ref, b1_ref, w2_ref, g2_ref):
    ssum = s1_ref[0, :N, :] + s1_ref[1, :N, :] + g1_ref[...]
    t = jnp.maximum(dinv_ref[...] * ssum + b1_ref[...], 0.0)
    g2_ref[...] = dinv_ref[...] * jnp.dot(
        t, w2_ref[...], preferred_element_type=jnp.float32)


_tc2 = pl.pallas_call(
    _tc2_body,
    out_shape=jax.ShapeDtypeStruct((N, H), jnp.float32),
)


def _tc3_body(s2_ref, g2_ref, dinv_ref, b2_ref, batch_ref, wfc_ref, bfc_ref,
              out_ref):
    ssum = s2_ref[0, :N, :] + s2_ref[1, :N, :] + g2_ref[...]
    o = jnp.maximum(dinv_ref[...] * ssum + b2_ref[...], 0.0)
    m = (batch_ref[...] == lax.broadcasted_iota(jnp.int32, (1, G), 1)
         ).astype(jnp.float32)
    sums = lax.dot_general(m, o, (((0,), (0,)), ((), ())),
                           preferred_element_type=jnp.float32)
    ones = jnp.ones((N, 1), jnp.float32)
    cnts = lax.dot_general(m, ones, (((0,), (0,)), ((), ())),
                           preferred_element_type=jnp.float32)
    pooled = sums / jnp.maximum(cnts, 1.0)
    out_ref[...] = jnp.dot(pooled, wfc_ref[...],
                           preferred_element_type=jnp.float32) + bfc_ref[...]


_tc3 = pl.pallas_call(
    _tc3_body,
    out_shape=jax.ShapeDtypeStruct((G, NCLS), jnp.float32),
)


@jax.jit
def kernel(x, edge_index, batch, W1, b1, W2, b2, Wfc, bfc):
    src = edge_index[0]
    dst = edge_index[1]
    srcr = jnp.concatenate(
        [src, jnp.zeros((EP - E,), jnp.int32)]).reshape(ROWS, LW)
    dstr = jnp.concatenate(
        [dst, jnp.full((EP - E,), N, jnp.int32)]).reshape(ROWS, LW)

    degp = _deg_kernel(dstr)
    degt = degp.T[:N]

    g1, dinv = _tc1(x, W1, degt)
    S1 = _conv_kernel(g1, srcr, dstr)
    g2 = _tc2(S1, g1, dinv, b1.reshape(1, H), W2)
    S2 = _conv_kernel(g2, srcr, dstr)
    out = _tc3(S2, g2, dinv, b2.reshape(1, H), batch.reshape(N, 1),
               Wfc, bfc.reshape(1, NCLS))
    return out


# speedup vs baseline: 1.4292x; 1.4292x over previous
import functools

import jax
import jax.numpy as jnp
from jax import lax
from jax.experimental import pallas as pl
from jax.experimental.pallas import tpu as pltpu
from jax.experimental.pallas import tpu_sc as plsc

N = 10000
E = 320000
DF = 128
H = 16
G = 64
NCLS = 2

NC = 2
NS = 16
NW = NC * NS
LW = 128
ROWS = E // LW
RPW = 78
CR = 6
CHUNKS = RPW // CR
TAIL = ROWS - NW * RPW
NPAD = 10112
RPS = NPAD // NS
DPAD = 10240
DPS = DPAD // NS

_sc_mesh = plsc.VectorSubcoreMesh(
    core_axis_name="c", subcore_axis_name="s", num_cores=NC, num_subcores=NS)

_sc_params = pltpu.CompilerParams(use_tc_tiling_on_sc=False)


def _deg_body(ei3, out, didx, ones_v, zbuf, dacc):
    c = lax.axis_index("c")
    s = lax.axis_index("s")
    w = c * NS + s

    def _zero(j, carry):
        zbuf[pl.ds(j * 16, 16)] = jnp.zeros((16,), jnp.float32)
        return carry

    lax.fori_loop(0, DPS // 16, _zero, 0)

    def _one(j, carry):
        ones_v[pl.ds(j * 16, 16)] = jnp.ones((16,), jnp.float32)
        return carry

    lax.fori_loop(0, LW // 16, _one, 0)

    pltpu.sync_copy(zbuf, dacc.at[pl.ds(s * DPS, DPS)])
    plsc.subcore_barrier()

    def _chunk(ci, carry):
        base = w * RPW + ci * CR
        pltpu.sync_copy(ei3.at[1, pl.ds(base, CR)], didx)
        for j in range(CR):
            pltpu.sync_copy(ones_v, dacc.at[didx.at[j]], add=True)
        return carry

    lax.fori_loop(0, CHUNKS, _chunk, 0)

    @pl.when(w >= NW - TAIL)
    def _tail():
        base = NW * RPW + (w - (NW - TAIL))
        pltpu.sync_copy(ei3.at[1, pl.ds(base, 1)], didx.at[pl.ds(0, 1)])
        pltpu.sync_copy(ones_v, dacc.at[didx.at[0]], add=True)

    plsc.subcore_barrier()
    pltpu.sync_copy(dacc.at[pl.ds(s * DPS, DPS)], out.at[c, pl.ds(s * DPS, DPS)])


_deg_kernel = pl.kernel(
    _deg_body,
    out_type=jax.ShapeDtypeStruct((NC, DPAD), jnp.float32),
    mesh=_sc_mesh,
    scratch_types=[
        pltpu.VMEM((CR, LW), jnp.int32),
        pltpu.VMEM((LW,), jnp.float32),
        pltpu.VMEM((DPS,), jnp.float32),
        pltpu.VMEM_SHARED((DPAD,), jnp.float32),
    ],
    compiler_params=_sc_params,
)


def _conv_body(g, ei3, out, sidx, didx, rows, zbuf, acc, sems):
    c = lax.axis_index("c")
    s = lax.axis_index("s")
    w = c * NS + s

    def _zero(j, carry):
        zbuf[j, :] = jnp.zeros((16,), jnp.float32)
        return carry

    lax.fori_loop(0, RPS, _zero, 0)
    pltpu.sync_copy(zbuf, acc.at[pl.ds(s * RPS, RPS)])
    plsc.subcore_barrier()

    def _chunk(ci, carry):
        base = w * RPW + ci * CR
        pltpu.sync_copy(ei3.at[0, pl.ds(base, CR)], sidx)
        pltpu.sync_copy(ei3.at[1, pl.ds(base, CR)], didx)
        copies = [
            pltpu.async_copy(g.at[sidx.at[j]],
                             rows.at[pl.ds(j * LW, LW)], sems.at[j])
            for j in range(CR)
        ]
        for j in range(CR):
            copies[j].wait()
            pltpu.sync_copy(rows.at[pl.ds(j * LW, LW)],
                            acc.at[didx.at[j]], add=True)
        return carry

    lax.fori_loop(0, CHUNKS, _chunk, 0)

    @pl.when(w >= NW - TAIL)
    def _tail():
        base = NW * RPW + (w - (NW - TAIL))
        pltpu.sync_copy(ei3.at[0, pl.ds(base, 1)], sidx.at[pl.ds(0, 1)])
        pltpu.sync_copy(ei3.at[1, pl.ds(base, 1)], didx.at[pl.ds(0, 1)])
        pltpu.async_copy(g.at[sidx.at[0]], rows.at[pl.ds(0, LW)],
                         sems.at[0]).wait()
        pltpu.sync_copy(rows.at[pl.ds(0, LW)], acc.at[didx.at[0]], add=True)

    plsc.subcore_barrier()
    pltpu.sync_copy(acc.at[pl.ds(s * RPS, RPS)], out.at[c, pl.ds(s * RPS, RPS)])


_conv_kernel = pl.kernel(
    _conv_body,
    out_type=jax.ShapeDtypeStruct((NC, NPAD, H), jnp.float32),
    mesh=_sc_mesh,
    scratch_types=[
        pltpu.VMEM((CR, LW), jnp.int32),
        pltpu.VMEM((CR, LW), jnp.int32),
        pltpu.VMEM((CR * LW, H), jnp.float32),
        pltpu.VMEM((RPS, H), jnp.float32),
        pltpu.VMEM_SHARED((NPAD, H), jnp.float32),
        pltpu.SemaphoreType.DMA((CR,)),
    ],
    compiler_params=_sc_params,
)


def _tcmm_body(x_ref, w1_ref, h_ref):
    h_ref[...] = jnp.dot(x_ref[...], w1_ref[...],
                         preferred_element_type=jnp.float32)


_tcmm = pl.pallas_call(
    _tcmm_body,
    out_shape=jax.ShapeDtypeStruct((N, H), jnp.float32),
)


def _tcscale_body(h_ref, degt_ref, g_ref, dinv_ref):
    deg = degt_ref[:, 0:1] + degt_ref[:, 1:2] + 1.0
    dinv = lax.rsqrt(deg)
    dinv_ref[...] = dinv
    g_ref[...] = h_ref[...] * dinv


_tcscale = pl.pallas_call(
    _tcscale_body,
    out_shape=(
        jax.ShapeDtypeStruct((N, H), jnp.float32),
        jax.ShapeDtypeStruct((N, 1), jnp.float32),
    ),
)


def _tc2_body(s1_ref, g1_ref, dinv_ref, b1_ref, w2_ref, g2_ref):
    ssum = s1_ref[0, :N, :] + s1_ref[1, :N, :] + g1_ref[...]
    t = jnp.maximum(dinv_ref[...] * ssum + b1_ref[...], 0.0)
    g2_ref[...] = dinv_ref[...] * jnp.dot(
        t, w2_ref[...], preferred_element_type=jnp.float32)


_tc2 = pl.pallas_call(
    _tc2_body,
    out_shape=jax.ShapeDtypeStruct((N, H), jnp.float32),
)


def _tc3_body(s2_ref, g2_ref, dinv_ref, b2_ref, batch_ref, wfc_ref, bfc_ref,
              out_ref):
    ssum = s2_ref[0, :N, :] + s2_ref[1, :N, :] + g2_ref[...]
    o = jnp.maximum(dinv_ref[...] * ssum + b2_ref[...], 0.0)
    m = (batch_ref[...] == lax.broadcasted_iota(jnp.int32, (1, G), 1)
         ).astype(jnp.float32)
    sums = lax.dot_general(m, o, (((0,), (0,)), ((), ())),
                           preferred_element_type=jnp.float32)
    ones = jnp.ones((N, 1), jnp.float32)
    cnts = lax.dot_general(m, ones, (((0,), (0,)), ((), ())),
                           preferred_element_type=jnp.float32)
    pooled = sums / jnp.maximum(cnts, 1.0)
    out_ref[...] = jnp.dot(pooled, wfc_ref[...],
                           preferred_element_type=jnp.float32) + bfc_ref[...]


_tc3 = pl.pallas_call(
    _tc3_body,
    out_shape=jax.ShapeDtypeStruct((G, NCLS), jnp.float32),
)


@jax.jit
def kernel(x, edge_index, batch, W1, b1, W2, b2, Wfc, bfc):
    ei3 = edge_index.reshape(2, ROWS, LW)

    degp = _deg_kernel(ei3)
    degt = degp.T[:N]

    h1 = _tcmm(x, W1)
    g1, dinv = _tcscale(h1, degt)
    S1 = _conv_kernel(g1, ei3)
    g2 = _tc2(S1, g1, dinv, b1.reshape(1, H), W2)
    S2 = _conv_kernel(g2, ei3)
    out = _tc3(S2, g2, dinv, b2.reshape(1, H), batch.reshape(N, 1),
               Wfc, bfc.reshape(1, NCLS))
    return out


# speedup vs baseline: 1.4652x; 1.0252x over previous
import functools

import jax
import jax.numpy as jnp
from jax import lax
from jax.experimental import pallas as pl
from jax.experimental.pallas import tpu as pltpu
from jax.experimental.pallas import tpu_sc as plsc

N = 10000
E = 320000
DF = 128
H = 16
G = 64
NCLS = 2

NC = 2
NS = 16
NW = NC * NS
LW = 128
ROWS = E // LW
RPW = 78
CR = 6
CHUNKS = RPW // CR
TAIL = ROWS - NW * RPW
NPAD = 10112
RPS = NPAD // NS
DPAD = 10240
DPS = DPAD // NS

_sc_mesh = plsc.VectorSubcoreMesh(
    core_axis_name="c", subcore_axis_name="s", num_cores=NC, num_subcores=NS)

_sc_params = pltpu.CompilerParams(use_tc_tiling_on_sc=False)


def _deg_body(ei3, out, didx, ones_v, zbuf, dacc):
    c = lax.axis_index("c")
    s = lax.axis_index("s")
    w = c * NS + s

    def _zero(j, carry):
        zbuf[pl.ds(j * 16, 16)] = jnp.zeros((16,), jnp.float32)
        return carry

    lax.fori_loop(0, DPS // 16, _zero, 0)

    def _one(j, carry):
        ones_v[pl.ds(j * 16, 16)] = jnp.ones((16,), jnp.float32)
        return carry

    lax.fori_loop(0, LW // 16, _one, 0)

    pltpu.sync_copy(zbuf, dacc.at[pl.ds(s * DPS, DPS)])
    plsc.subcore_barrier()

    def _chunk(ci, carry):
        base = w * RPW + ci * CR
        pltpu.sync_copy(ei3.at[1, pl.ds(base, CR)], didx)
        for j in range(CR):
            pltpu.sync_copy(ones_v, dacc.at[didx.at[j]], add=True)
        return carry

    lax.fori_loop(0, CHUNKS, _chunk, 0)

    @pl.when(w >= NW - TAIL)
    def _tail():
        base = NW * RPW + (w - (NW - TAIL))
        pltpu.sync_copy(ei3.at[1, pl.ds(base, 1)], didx.at[pl.ds(0, 1)])
        pltpu.sync_copy(ones_v, dacc.at[didx.at[0]], add=True)

    plsc.subcore_barrier()
    pltpu.sync_copy(dacc.at[pl.ds(s * DPS, DPS)], out.at[c, pl.ds(s * DPS, DPS)])


_deg_kernel = pl.kernel(
    _deg_body,
    out_type=jax.ShapeDtypeStruct((NC, DPAD), jnp.float32),
    mesh=_sc_mesh,
    scratch_types=[
        pltpu.VMEM((CR, LW), jnp.int32),
        pltpu.VMEM((LW,), jnp.float32),
        pltpu.VMEM((DPS,), jnp.float32),
        pltpu.VMEM_SHARED((DPAD,), jnp.float32),
    ],
    compiler_params=_sc_params,
)


def _conv_body(g, ei3, out, sidx, didx, rows, zbuf, acc, sems):
    c = lax.axis_index("c")
    s = lax.axis_index("s")
    w = c * NS + s

    def _zero(j, carry):
        zbuf[pl.ds(2 * j, 2), :] = jnp.zeros((2, 16), jnp.bfloat16)
        return carry

    lax.fori_loop(0, RPS // 2, _zero, 0)
    pltpu.sync_copy(zbuf, acc.at[pl.ds(s * RPS, RPS)])
    plsc.subcore_barrier()

    def _chunk(ci, carry):
        base = w * RPW + ci * CR
        pltpu.sync_copy(ei3.at[0, pl.ds(base, CR)], sidx)
        pltpu.sync_copy(ei3.at[1, pl.ds(base, CR)], didx)
        copies = [
            pltpu.async_copy(g.at[sidx.at[j]],
                             rows.at[pl.ds(j * LW, LW)], sems.at[j])
            for j in range(CR)
        ]
        for j in range(CR):
            copies[j].wait()
            pltpu.sync_copy(rows.at[pl.ds(j * LW, LW)],
                            acc.at[didx.at[j]], add=True)
        return carry

    lax.fori_loop(0, CHUNKS, _chunk, 0)

    @pl.when(w >= NW - TAIL)
    def _tail():
        base = NW * RPW + (w - (NW - TAIL))
        pltpu.sync_copy(ei3.at[0, pl.ds(base, 1)], sidx.at[pl.ds(0, 1)])
        pltpu.sync_copy(ei3.at[1, pl.ds(base, 1)], didx.at[pl.ds(0, 1)])
        pltpu.async_copy(g.at[sidx.at[0]], rows.at[pl.ds(0, LW)],
                         sems.at[0]).wait()
        pltpu.sync_copy(rows.at[pl.ds(0, LW)], acc.at[didx.at[0]], add=True)

    plsc.subcore_barrier()
    pltpu.sync_copy(acc.at[pl.ds(s * RPS, RPS)], out.at[c, pl.ds(s * RPS, RPS)])


_conv_kernel = pl.kernel(
    _conv_body,
    out_type=jax.ShapeDtypeStruct((NC, NPAD, H), jnp.bfloat16),
    mesh=_sc_mesh,
    scratch_types=[
        pltpu.VMEM((CR, LW), jnp.int32),
        pltpu.VMEM((CR, LW), jnp.int32),
        pltpu.VMEM((CR * LW, H), jnp.bfloat16),
        pltpu.VMEM((RPS, H), jnp.bfloat16),
        pltpu.VMEM_SHARED((NPAD, H), jnp.bfloat16),
        pltpu.SemaphoreType.DMA((CR,)),
    ],
    compiler_params=_sc_params,
)


def _tcmm_body(x_ref, w1_ref, h_ref):
    h_ref[...] = jnp.dot(x_ref[...], w1_ref[...],
                         preferred_element_type=jnp.float32)


_tcmm = pl.pallas_call(
    _tcmm_body,
    out_shape=jax.ShapeDtypeStruct((N, H), jnp.float32),
)


def _tcscale_body(h_ref, degt_ref, g_ref, gb_ref, dinv_ref):
    deg = degt_ref[:, 0:1] + degt_ref[:, 1:2] + 1.0
    dinv = lax.rsqrt(deg)
    dinv_ref[...] = dinv
    g = h_ref[...] * dinv
    g_ref[...] = g
    gb_ref[...] = g.astype(jnp.bfloat16)


_tcscale = pl.pallas_call(
    _tcscale_body,
    out_shape=(
        jax.ShapeDtypeStruct((N, H), jnp.float32),
        jax.ShapeDtypeStruct((N, H), jnp.bfloat16),
        jax.ShapeDtypeStruct((N, 1), jnp.float32),
    ),
)


def _tc2_body(s1_ref, g1_ref, dinv_ref, b1_ref, w2_ref, g2_ref, g2b_ref):
    ssum = (s1_ref[0, :N, :] + s1_ref[1, :N, :]).astype(jnp.float32) \
        + g1_ref[...]
    t = jnp.maximum(dinv_ref[...] * ssum + b1_ref[...], 0.0)
    g2 = dinv_ref[...] * jnp.dot(
        t, w2_ref[...], preferred_element_type=jnp.float32)
    g2_ref[...] = g2
    g2b_ref[...] = g2.astype(jnp.bfloat16)


_tc2 = pl.pallas_call(
    _tc2_body,
    out_shape=(
        jax.ShapeDtypeStruct((N, H), jnp.float32),
        jax.ShapeDtypeStruct((N, H), jnp.bfloat16),
    ),
)


def _tc3_body(s2_ref, g2_ref, dinv_ref, b2_ref, batch_ref, wfc_ref, bfc_ref,
              out_ref):
    ssum = (s2_ref[0, :N, :] + s2_ref[1, :N, :]).astype(jnp.float32) \
        + g2_ref[...]
    o = jnp.maximum(dinv_ref[...] * ssum + b2_ref[...], 0.0)
    m = (batch_ref[...] == lax.broadcasted_iota(jnp.int32, (1, G), 1)
         ).astype(jnp.float32)
    sums = lax.dot_general(m, o, (((0,), (0,)), ((), ())),
                           preferred_element_type=jnp.float32)
    ones = jnp.ones((N, 1), jnp.float32)
    cnts = lax.dot_general(m, ones, (((0,), (0,)), ((), ())),
                           preferred_element_type=jnp.float32)
    pooled = sums / jnp.maximum(cnts, 1.0)
    out_ref[...] = jnp.dot(pooled, wfc_ref[...],
                           preferred_element_type=jnp.float32) + bfc_ref[...]


_tc3 = pl.pallas_call(
    _tc3_body,
    out_shape=jax.ShapeDtypeStruct((G, NCLS), jnp.float32),
)


@jax.jit
def kernel(x, edge_index, batch, W1, b1, W2, b2, Wfc, bfc):
    ei3 = edge_index.reshape(2, ROWS, LW)

    degp = _deg_kernel(ei3)
    degt = degp.T[:N]

    h1 = _tcmm(x, W1)
    g1, g1b, dinv = _tcscale(h1, degt)
    S1 = _conv_kernel(g1b, ei3)
    g2, g2b = _tc2(S1, g1, dinv, b1.reshape(1, H), W2)
    S2 = _conv_kernel(g2b, ei3)
    out = _tc3(S2, g2, dinv, b2.reshape(1, H), batch.reshape(N, 1),
               Wfc, bfc.reshape(1, NCLS))
    return out


# speedup vs baseline: 1.5124x; 1.0322x over previous
import functools

import jax
import jax.numpy as jnp
from jax import lax
from jax.experimental import pallas as pl
from jax.experimental.pallas import tpu as pltpu
from jax.experimental.pallas import tpu_sc as plsc

N = 10000
E = 320000
DF = 128
H = 16
G = 64
NCLS = 2

NC = 2
NS = 16
NW = NC * NS
LW = 128
ROWS = E // LW
RPW = 78
CR = 6
CHUNKS = RPW // CR
TAIL = ROWS - NW * RPW
NPAD = 10112
RPS = NPAD // NS
DPAD = 10240
DPS = DPAD // NS

_sc_mesh = plsc.VectorSubcoreMesh(
    core_axis_name="c", subcore_axis_name="s", num_cores=NC, num_subcores=NS)

_sc_params = pltpu.CompilerParams(use_tc_tiling_on_sc=False)


def _deg_body(ei3, out, didx, ones_v, zbuf, dacc):
    c = lax.axis_index("c")
    s = lax.axis_index("s")
    w = c * NS + s

    def _zero(j, carry):
        zbuf[pl.ds(j * 16, 16)] = jnp.zeros((16,), jnp.float32)
        return carry

    lax.fori_loop(0, DPS // 16, _zero, 0)

    def _one(j, carry):
        ones_v[pl.ds(j * 16, 16)] = jnp.ones((16,), jnp.float32)
        return carry

    lax.fori_loop(0, LW // 16, _one, 0)

    pltpu.sync_copy(zbuf, dacc.at[pl.ds(s * DPS, DPS)])
    plsc.subcore_barrier()

    def _chunk(ci, carry):
        base = w * RPW + ci * CR
        pltpu.sync_copy(ei3.at[1, pl.ds(base, CR)], didx)
        for j in range(CR):
            pltpu.sync_copy(ones_v, dacc.at[didx.at[j]], add=True)
        return carry

    lax.fori_loop(0, CHUNKS, _chunk, 0)

    @pl.when(w >= NW - TAIL)
    def _tail():
        base = NW * RPW + (w - (NW - TAIL))
        pltpu.sync_copy(ei3.at[1, pl.ds(base, 1)], didx.at[pl.ds(0, 1)])
        pltpu.sync_copy(ones_v, dacc.at[didx.at[0]], add=True)

    plsc.subcore_barrier()
    pltpu.sync_copy(dacc.at[pl.ds(s * DPS, DPS)], out.at[c, pl.ds(s * DPS, DPS)])


_deg_kernel = pl.kernel(
    _deg_body,
    out_type=jax.ShapeDtypeStruct((NC, DPAD), jnp.float32),
    mesh=_sc_mesh,
    scratch_types=[
        pltpu.VMEM((CR, LW), jnp.int32),
        pltpu.VMEM((LW,), jnp.float32),
        pltpu.VMEM((DPS,), jnp.float32),
        pltpu.VMEM_SHARED((DPAD,), jnp.float32),
    ],
    compiler_params=_sc_params,
)


def _conv_body(g, ei3, out, sidx, didx, rows, zbuf, acc, sems, ssem):
    c = lax.axis_index("c")
    s = lax.axis_index("s")
    w = c * NS + s

    def _zero(j, carry):
        zbuf[pl.ds(2 * j, 2), :] = jnp.zeros((2, 16), jnp.bfloat16)
        return carry

    lax.fori_loop(0, RPS // 2, _zero, 0)
    pltpu.sync_copy(zbuf, acc.at[pl.ds(s * RPS, RPS)])
    plsc.subcore_barrier()

    def _drain_scatters():
        for j in range(CR):
            pltpu.make_async_copy(rows.at[pl.ds(j * LW, LW)],
                                  acc.at[didx.at[j]], ssem).wait()

    def _chunk(ci, carry):
        base = w * RPW + ci * CR
        @pl.when(ci > 0)
        def _():
            _drain_scatters()

        pltpu.sync_copy(ei3.at[0, pl.ds(base, CR)], sidx)
        pltpu.sync_copy(ei3.at[1, pl.ds(base, CR)], didx)
        copies = [
            pltpu.async_copy(g.at[sidx.at[j]],
                             rows.at[pl.ds(j * LW, LW)], sems.at[j])
            for j in range(CR)
        ]
        for j in range(CR):
            copies[j].wait()
            pltpu.async_copy(rows.at[pl.ds(j * LW, LW)],
                             acc.at[didx.at[j]], ssem, add=True)
        return carry

    lax.fori_loop(0, CHUNKS, _chunk, 0)
    _drain_scatters()

    @pl.when(w >= NW - TAIL)
    def _tail():
        base = NW * RPW + (w - (NW - TAIL))
        pltpu.sync_copy(ei3.at[0, pl.ds(base, 1)], sidx.at[pl.ds(0, 1)])
        pltpu.sync_copy(ei3.at[1, pl.ds(base, 1)], didx.at[pl.ds(0, 1)])
        pltpu.async_copy(g.at[sidx.at[0]], rows.at[pl.ds(0, LW)],
                         sems.at[0]).wait()
        pltpu.sync_copy(rows.at[pl.ds(0, LW)], acc.at[didx.at[0]], add=True)

    plsc.subcore_barrier()
    pltpu.sync_copy(acc.at[pl.ds(s * RPS, RPS)], out.at[c, pl.ds(s * RPS, RPS)])


_conv_kernel = pl.kernel(
    _conv_body,
    out_type=jax.ShapeDtypeStruct((NC, NPAD, H), jnp.bfloat16),
    mesh=_sc_mesh,
    scratch_types=[
        pltpu.VMEM((CR, LW), jnp.int32),
        pltpu.VMEM((CR, LW), jnp.int32),
        pltpu.VMEM((CR * LW, H), jnp.bfloat16),
        pltpu.VMEM((RPS, H), jnp.bfloat16),
        pltpu.VMEM_SHARED((NPAD, H), jnp.bfloat16),
        pltpu.SemaphoreType.DMA((CR,)),
        pltpu.SemaphoreType.DMA,
    ],
    compiler_params=_sc_params,
)


def _tcmm_body(x_ref, w1_ref, h_ref):
    h_ref[...] = jnp.dot(x_ref[...], w1_ref[...],
                         preferred_element_type=jnp.float32)


_tcmm = pl.pallas_call(
    _tcmm_body,
    out_shape=jax.ShapeDtypeStruct((N, H), jnp.float32),
)


def _tcscale_body(h_ref, degt_ref, g_ref, gb_ref, dinv_ref):
    deg = degt_ref[:, 0:1] + degt_ref[:, 1:2] + 1.0
    dinv = lax.rsqrt(deg)
    dinv_ref[...] = dinv
    g = h_ref[...] * dinv
    g_ref[...] = g
    gb_ref[...] = g.astype(jnp.bfloat16)


_tcscale = pl.pallas_call(
    _tcscale_body,
    out_shape=(
        jax.ShapeDtypeStruct((N, H), jnp.float32),
        jax.ShapeDtypeStruct((N, H), jnp.bfloat16),
        jax.ShapeDtypeStruct((N, 1), jnp.float32),
    ),
)


def _tc2_body(s1_ref, g1_ref, dinv_ref, b1_ref, w2_ref, g2_ref, g2b_ref):
    ssum = (s1_ref[0, :N, :] + s1_ref[1, :N, :]).astype(jnp.float32) \
        + g1_ref[...]
    t = jnp.maximum(dinv_ref[...] * ssum + b1_ref[...], 0.0)
    g2 = dinv_ref[...] * jnp.dot(
        t, w2_ref[...], preferred_element_type=jnp.float32)
    g2_ref[...] = g2
    g2b_ref[...] = g2.astype(jnp.bfloat16)


_tc2 = pl.pallas_call(
    _tc2_body,
    out_shape=(
        jax.ShapeDtypeStruct((N, H), jnp.float32),
        jax.ShapeDtypeStruct((N, H), jnp.bfloat16),
    ),
)


def _tc3_body(s2_ref, g2_ref, dinv_ref, b2_ref, batch_ref, wfc_ref, bfc_ref,
              out_ref):
    ssum = (s2_ref[0, :N, :] + s2_ref[1, :N, :]).astype(jnp.float32) \
        + g2_ref[...]
    o = jnp.maximum(dinv_ref[...] * ssum + b2_ref[...], 0.0)
    m = (batch_ref[...] == lax.broadcasted_iota(jnp.int32, (1, G), 1)
         ).astype(jnp.float32)
    sums = lax.dot_general(m, o, (((0,), (0,)), ((), ())),
                           preferred_element_type=jnp.float32)
    ones = jnp.ones((N, 1), jnp.float32)
    cnts = lax.dot_general(m, ones, (((0,), (0,)), ((), ())),
                           preferred_element_type=jnp.float32)
    pooled = sums / jnp.maximum(cnts, 1.0)
    out_ref[...] = jnp.dot(pooled, wfc_ref[...],
                           preferred_element_type=jnp.float32) + bfc_ref[...]


_tc3 = pl.pallas_call(
    _tc3_body,
    out_shape=jax.ShapeDtypeStruct((G, NCLS), jnp.float32),
)


@jax.jit
def kernel(x, edge_index, batch, W1, b1, W2, b2, Wfc, bfc):
    ei3 = edge_index.reshape(2, ROWS, LW)

    degp = _deg_kernel(ei3)
    degt = degp.T[:N]

    h1 = _tcmm(x, W1)
    g1, g1b, dinv = _tcscale(h1, degt)
    S1 = _conv_kernel(g1b, ei3)
    g2, g2b = _tc2(S1, g1, dinv, b1.reshape(1, H), W2)
    S2 = _conv_kernel(g2b, ei3)
    out = _tc3(S2, g2, dinv, b2.reshape(1, H), batch.reshape(N, 1),
               Wfc, bfc.reshape(1, NCLS))
    return out


# speedup vs baseline: 1.7400x; 1.1505x over previous
import functools

import jax
import jax.numpy as jnp
from jax import lax
from jax.experimental import pallas as pl
from jax.experimental.pallas import tpu as pltpu
from jax.experimental.pallas import tpu_sc as plsc

N = 10000
E = 320000
DF = 128
H = 16
G = 64
NCLS = 2

NC = 2
NS = 16
NW = NC * NS
LW = 128
ROWS = E // LW
RPW = 78
CR = 6
CHUNKS = RPW // CR
TAIL = ROWS - NW * RPW
NPAD = 10112
RPS = NPAD // NS
DPAD = 10240
DPS = DPAD // NS

_sc_mesh = plsc.VectorSubcoreMesh(
    core_axis_name="c", subcore_axis_name="s", num_cores=NC, num_subcores=NS)

_sc_params = pltpu.CompilerParams(use_tc_tiling_on_sc=False)


def _deg_body(ei3, out, didx, ones_v, zbuf, dacc):
    c = lax.axis_index("c")
    s = lax.axis_index("s")
    w = c * NS + s

    def _zero(j, carry):
        zbuf[pl.ds(j * 16, 16)] = jnp.zeros((16,), jnp.float32)
        return carry

    lax.fori_loop(0, DPS // 16, _zero, 0)

    def _one(j, carry):
        ones_v[pl.ds(j * 16, 16)] = jnp.ones((16,), jnp.float32)
        return carry

    lax.fori_loop(0, LW // 16, _one, 0)

    pltpu.sync_copy(zbuf, dacc.at[pl.ds(s * DPS, DPS)])
    plsc.subcore_barrier()

    def _chunk(ci, carry):
        base = w * RPW + ci * CR
        pltpu.sync_copy(ei3.at[1, pl.ds(base, CR)], didx)
        for j in range(CR):
            pltpu.sync_copy(ones_v, dacc.at[didx.at[j]], add=True)
        return carry

    lax.fori_loop(0, CHUNKS, _chunk, 0)

    @pl.when(w >= NW - TAIL)
    def _tail():
        base = NW * RPW + (w - (NW - TAIL))
        pltpu.sync_copy(ei3.at[1, pl.ds(base, 1)], didx.at[pl.ds(0, 1)])
        pltpu.sync_copy(ones_v, dacc.at[didx.at[0]], add=True)

    plsc.subcore_barrier()
    pltpu.sync_copy(dacc.at[pl.ds(s * DPS, DPS)], out.at[c, pl.ds(s * DPS, DPS)])


_deg_kernel = pl.kernel(
    _deg_body,
    out_type=jax.ShapeDtypeStruct((NC, DPAD), jnp.float32),
    mesh=_sc_mesh,
    scratch_types=[
        pltpu.VMEM((CR, LW), jnp.int32),
        pltpu.VMEM((LW,), jnp.float32),
        pltpu.VMEM((DPS,), jnp.float32),
        pltpu.VMEM_SHARED((DPAD,), jnp.float32),
    ],
    compiler_params=_sc_params,
)


def _conv_body(g, ei3, out, sall, dall, rows0, rows1, zbuf, acc, sems,
               ssem0, ssem1):
    c = lax.axis_index("c")
    s = lax.axis_index("s")
    w = c * NS + s

    def _zero(j, carry):
        zbuf[pl.ds(2 * j, 2), :] = jnp.zeros((2, 16), jnp.bfloat16)
        return carry

    lax.fori_loop(0, RPS // 2, _zero, 0)
    pltpu.sync_copy(zbuf, acc.at[pl.ds(s * RPS, RPS)])

    pltpu.sync_copy(ei3.at[0, pl.ds(w * RPW, RPW)], sall.at[pl.ds(0, RPW)])
    pltpu.sync_copy(ei3.at[1, pl.ds(w * RPW, RPW)], dall.at[pl.ds(0, RPW)])

    @pl.when(w >= NW - TAIL)
    def _tail_idx():
        base = NW * RPW + (w - (NW - TAIL))
        pltpu.sync_copy(ei3.at[0, pl.ds(base, 1)], sall.at[pl.ds(RPW, 1)])
        pltpu.sync_copy(ei3.at[1, pl.ds(base, 1)], dall.at[pl.ds(RPW, 1)])

    plsc.subcore_barrier()

    def _drain(rows, ssem):
        for j in range(CR):
            pltpu.make_async_copy(rows.at[pl.ds(j * LW, LW)],
                                  acc.at[dall.at[0]], ssem).wait()

    def _run_chunk(ci, rows, ssem):
        copies = [
            pltpu.async_copy(g.at[sall.at[ci * CR + j]],
                             rows.at[pl.ds(j * LW, LW)], sems.at[j])
            for j in range(CR)
        ]
        for j in range(CR):
            copies[j].wait()
            pltpu.async_copy(rows.at[pl.ds(j * LW, LW)],
                             acc.at[dall.at[ci * CR + j]], ssem, add=True)

    def _pair(i, carry):
        @pl.when(i > 0)
        def _():
            _drain(rows0, ssem0)
        _run_chunk(2 * i, rows0, ssem0)

        @pl.when(i > 0)
        def _():
            _drain(rows1, ssem1)
        _run_chunk(2 * i + 1, rows1, ssem1)
        return carry

    lax.fori_loop(0, CHUNKS // 2, _pair, 0)
    _drain(rows0, ssem0)
    _run_chunk(CHUNKS - 1, rows0, ssem0)
    _drain(rows0, ssem0)
    _drain(rows1, ssem1)

    @pl.when(w >= NW - TAIL)
    def _tail():
        pltpu.async_copy(g.at[sall.at[RPW]], rows0.at[pl.ds(0, LW)],
                         sems.at[0]).wait()
        pltpu.sync_copy(rows0.at[pl.ds(0, LW)], acc.at[dall.at[RPW]],
                        add=True)

    plsc.subcore_barrier()
    pltpu.sync_copy(acc.at[pl.ds(s * RPS, RPS)], out.at[c, pl.ds(s * RPS, RPS)])


_conv_kernel = pl.kernel(
    _conv_body,
    out_type=jax.ShapeDtypeStruct((NC, NPAD, H), jnp.bfloat16),
    mesh=_sc_mesh,
    scratch_types=[
        pltpu.VMEM((RPW + 1, LW), jnp.int32),
        pltpu.VMEM((RPW + 1, LW), jnp.int32),
        pltpu.VMEM((CR * LW, H), jnp.bfloat16),
        pltpu.VMEM((CR * LW, H), jnp.bfloat16),
        pltpu.VMEM((RPS, H), jnp.bfloat16),
        pltpu.VMEM_SHARED((NPAD, H), jnp.bfloat16),
        pltpu.SemaphoreType.DMA((CR,)),
        pltpu.SemaphoreType.DMA,
        pltpu.SemaphoreType.DMA,
    ],
    compiler_params=_sc_params,
)


def _tcmm_body(x_ref, w1_ref, h_ref):
    h_ref[...] = jnp.dot(x_ref[...], w1_ref[...],
                         preferred_element_type=jnp.float32)


_tcmm = pl.pallas_call(
    _tcmm_body,
    out_shape=jax.ShapeDtypeStruct((N, H), jnp.float32),
)


def _tcscale_body(h_ref, degt_ref, g_ref, gb_ref, dinv_ref):
    deg = degt_ref[:, 0:1] + degt_ref[:, 1:2] + 1.0
    dinv = lax.rsqrt(deg)
    dinv_ref[...] = dinv
    g = h_ref[...] * dinv
    g_ref[...] = g
    gb_ref[...] = g.astype(jnp.bfloat16)


_tcscale = pl.pallas_call(
    _tcscale_body,
    out_shape=(
        jax.ShapeDtypeStruct((N, H), jnp.float32),
        jax.ShapeDtypeStruct((N, H), jnp.bfloat16),
        jax.ShapeDtypeStruct((N, 1), jnp.float32),
    ),
)


def _tc2_body(s1_ref, g1_ref, dinv_ref, b1_ref, w2_ref, g2_ref, g2b_ref):
    ssum = (s1_ref[0, :N, :] + s1_ref[1, :N, :]).astype(jnp.float32) \
        + g1_ref[...]
    t = jnp.maximum(dinv_ref[...] * ssum + b1_ref[...], 0.0)
    g2 = dinv_ref[...] * jnp.dot(
        t, w2_ref[...], preferred_element_type=jnp.float32)
    g2_ref[...] = g2
    g2b_ref[...] = g2.astype(jnp.bfloat16)


_tc2 = pl.pallas_call(
    _tc2_body,
    out_shape=(
        jax.ShapeDtypeStruct((N, H), jnp.float32),
        jax.ShapeDtypeStruct((N, H), jnp.bfloat16),
    ),
)


def _tc3_body(s2_ref, g2_ref, dinv_ref, b2_ref, batch_ref, wfc_ref, bfc_ref,
              out_ref):
    ssum = (s2_ref[0, :N, :] + s2_ref[1, :N, :]).astype(jnp.float32) \
        + g2_ref[...]
    o = jnp.maximum(dinv_ref[...] * ssum + b2_ref[...], 0.0)
    m = (batch_ref[...] == lax.broadcasted_iota(jnp.int32, (1, G), 1)
         ).astype(jnp.float32)
    sums = lax.dot_general(m, o, (((0,), (0,)), ((), ())),
                           preferred_element_type=jnp.float32)
    ones = jnp.ones((N, 1), jnp.float32)
    cnts = lax.dot_general(m, ones, (((0,), (0,)), ((), ())),
                           preferred_element_type=jnp.float32)
    pooled = sums / jnp.maximum(cnts, 1.0)
    out_ref[...] = jnp.dot(pooled, wfc_ref[...],
                           preferred_element_type=jnp.float32) + bfc_ref[...]


_tc3 = pl.pallas_call(
    _tc3_body,
    out_shape=jax.ShapeDtypeStruct((G, NCLS), jnp.float32),
)


@jax.jit
def kernel(x, edge_index, batch, W1, b1, W2, b2, Wfc, bfc):
    ei3 = edge_index.reshape(2, ROWS, LW)

    degp = _deg_kernel(ei3)
    degt = degp.T[:N]

    h1 = _tcmm(x, W1)
    g1, g1b, dinv = _tcscale(h1, degt)
    S1 = _conv_kernel(g1b, ei3)
    g2, g2b = _tc2(S1, g1, dinv, b1.reshape(1, H), W2)
    S2 = _conv_kernel(g2b, ei3)
    out = _tc3(S2, g2, dinv, b2.reshape(1, H), batch.reshape(N, 1),
               Wfc, bfc.reshape(1, NCLS))
    return out


# speedup vs baseline: 2.3254x; 1.3364x over previous
import functools

import jax
import jax.numpy as jnp
from jax import lax
from jax.experimental import pallas as pl
from jax.experimental.pallas import tpu as pltpu
from jax.experimental.pallas import tpu_sc as plsc

N = 10000
E = 320000
DF = 128
H = 16
G = 64
NCLS = 2

NC = 2
NS = 16
NW = NC * NS
LW = 128
ROWS = E // LW
RPW = 78
CR = 6
CHUNKS = RPW // CR
TAIL = ROWS - NW * RPW
NPAD = 10112
RPS = NPAD // NS
DPAD = 10240
DPS = DPAD // NS

_sc_mesh = plsc.VectorSubcoreMesh(
    core_axis_name="c", subcore_axis_name="s", num_cores=NC, num_subcores=NS)

_sc_params = pltpu.CompilerParams(use_tc_tiling_on_sc=False)


def _deg_body(ei3, out, didx, ones_v, zbuf, dacc):
    c = lax.axis_index("c")
    s = lax.axis_index("s")
    w = c * NS + s

    def _zero(j, carry):
        zbuf[pl.ds(j * 16, 16)] = jnp.zeros((16,), jnp.float32)
        return carry

    lax.fori_loop(0, DPS // 16, _zero, 0)

    def _one(j, carry):
        ones_v[pl.ds(j * 16, 16)] = jnp.ones((16,), jnp.float32)
        return carry

    lax.fori_loop(0, LW // 16, _one, 0)

    pltpu.sync_copy(zbuf, dacc.at[pl.ds(s * DPS, DPS)])
    plsc.subcore_barrier()

    def _chunk(ci, carry):
        base = w * RPW + ci * CR
        pltpu.sync_copy(ei3.at[1, pl.ds(base, CR)], didx)
        for j in range(CR):
            pltpu.sync_copy(ones_v, dacc.at[didx.at[j]], add=True)
        return carry

    lax.fori_loop(0, CHUNKS, _chunk, 0)

    @pl.when(w >= NW - TAIL)
    def _tail():
        base = NW * RPW + (w - (NW - TAIL))
        pltpu.sync_copy(ei3.at[1, pl.ds(base, 1)], didx.at[pl.ds(0, 1)])
        pltpu.sync_copy(ones_v, dacc.at[didx.at[0]], add=True)

    plsc.subcore_barrier()
    pltpu.sync_copy(dacc.at[pl.ds(s * DPS, DPS)], out.at[c, pl.ds(s * DPS, DPS)])


_deg_kernel = pl.kernel(
    _deg_body,
    out_type=jax.ShapeDtypeStruct((NC, DPAD), jnp.float32),
    mesh=_sc_mesh,
    scratch_types=[
        pltpu.VMEM((CR, LW), jnp.int32),
        pltpu.VMEM((LW,), jnp.float32),
        pltpu.VMEM((DPS,), jnp.float32),
        pltpu.VMEM_SHARED((DPAD,), jnp.float32),
    ],
    compiler_params=_sc_params,
)


def _conv_body(g, ei3, out, sall, dall, rows0, rows1, zbuf, acc, sems,
               ssem0, ssem1):
    c = lax.axis_index("c")
    s = lax.axis_index("s")
    w = c * NS + s

    def _zero(j, carry):
        zbuf[pl.ds(2 * j, 2), :] = jnp.zeros((2, 16), jnp.bfloat16)
        return carry

    lax.fori_loop(0, RPS // 2, _zero, 0)
    pltpu.sync_copy(zbuf, acc.at[pl.ds(s * RPS, RPS)])

    pltpu.sync_copy(ei3.at[0, pl.ds(w * RPW, RPW)], sall.at[pl.ds(0, RPW)])
    pltpu.sync_copy(ei3.at[1, pl.ds(w * RPW, RPW)], dall.at[pl.ds(0, RPW)])

    @pl.when(w >= NW - TAIL)
    def _tail_idx():
        base = NW * RPW + (w - (NW - TAIL))
        pltpu.sync_copy(ei3.at[0, pl.ds(base, 1)], sall.at[pl.ds(RPW, 1)])
        pltpu.sync_copy(ei3.at[1, pl.ds(base, 1)], dall.at[pl.ds(RPW, 1)])

    plsc.subcore_barrier()

    def _drain(rows, ssem):
        for j in range(CR):
            pltpu.make_async_copy(rows.at[pl.ds(j * LW, LW)],
                                  acc.at[dall.at[0]], ssem).wait()

    def _run_chunk(ci, rows, ssem):
        copies = [
            pltpu.async_copy(g.at[sall.at[ci * CR + j]],
                             rows.at[pl.ds(j * LW, LW)], sems.at[j])
            for j in range(CR)
        ]
        for j in range(CR):
            copies[j].wait()
            pltpu.async_copy(rows.at[pl.ds(j * LW, LW)],
                             acc.at[dall.at[ci * CR + j]], ssem, add=True)

    def _pair(i, carry):
        @pl.when(i > 0)
        def _():
            _drain(rows0, ssem0)
        _run_chunk(2 * i, rows0, ssem0)

        @pl.when(i > 0)
        def _():
            _drain(rows1, ssem1)
        _run_chunk(2 * i + 1, rows1, ssem1)
        return carry

    lax.fori_loop(0, CHUNKS // 2, _pair, 0)
    _drain(rows0, ssem0)
    _run_chunk(CHUNKS - 1, rows0, ssem0)
    _drain(rows0, ssem0)
    _drain(rows1, ssem1)

    @pl.when(w >= NW - TAIL)
    def _tail():
        pltpu.async_copy(g.at[sall.at[RPW]], rows0.at[pl.ds(0, LW)],
                         sems.at[0]).wait()
        pltpu.sync_copy(rows0.at[pl.ds(0, LW)], acc.at[dall.at[RPW]],
                        add=True)

    plsc.subcore_barrier()
    pltpu.sync_copy(acc.at[pl.ds(s * RPS, RPS)], out.at[c, pl.ds(s * RPS, RPS)])


_conv_kernel = pl.kernel(
    _conv_body,
    out_type=jax.ShapeDtypeStruct((NC, NPAD, H), jnp.bfloat16),
    mesh=_sc_mesh,
    scratch_types=[
        pltpu.VMEM((RPW + 1, LW), jnp.int32),
        pltpu.VMEM((RPW + 1, LW), jnp.int32),
        pltpu.VMEM((CR * LW, H), jnp.bfloat16),
        pltpu.VMEM((CR * LW, H), jnp.bfloat16),
        pltpu.VMEM((RPS, H), jnp.bfloat16),
        pltpu.VMEM_SHARED((NPAD, H), jnp.bfloat16),
        pltpu.SemaphoreType.DMA((CR,)),
        pltpu.SemaphoreType.DMA,
        pltpu.SemaphoreType.DMA,
    ],
    compiler_params=_sc_params,
)


PR = N // 8
SR = NPAD // 8


def _tcmm_body(x_ref, w1bd_ref, h_ref):
    h_ref[...] = jnp.dot(x_ref[...], w1bd_ref[...],
                         preferred_element_type=jnp.float32)


_tcmm = pl.pallas_call(
    _tcmm_body,
    out_shape=jax.ShapeDtypeStruct((PR, 128), jnp.float32),
)


def _tcdinv_body(degp_ref, dinv_ref):
    deg = degp_ref[0] + degp_ref[1] + 1.0
    dinv_ref[...] = lax.rsqrt(deg)


_tcdinv = pl.pallas_call(
    _tcdinv_body,
    out_shape=jax.ShapeDtypeStruct((DPAD // 128, 128), jnp.float32),
)


def _tcscale_body(h_ref, dinvp_ref, g_ref, gb_ref):
    g = h_ref[...] * dinvp_ref[...]
    g_ref[...] = g
    gb_ref[...] = g.astype(jnp.bfloat16)


_tcscale = pl.pallas_call(
    _tcscale_body,
    out_shape=(
        jax.ShapeDtypeStruct((PR, 128), jnp.float32),
        jax.ShapeDtypeStruct((PR, 128), jnp.bfloat16),
    ),
)


def _tc2_body(s1_ref, g1_ref, dinvp_ref, b1t_ref, w2bd_ref, g2_ref, g2b_ref):
    ssum = (s1_ref[0, :PR, :] + s1_ref[1, :PR, :]).astype(jnp.float32) \
        + g1_ref[...]
    t = jnp.maximum(dinvp_ref[...] * ssum + b1t_ref[...], 0.0)
    g2 = dinvp_ref[...] * jnp.dot(
        t, w2bd_ref[...], preferred_element_type=jnp.float32)
    g2_ref[...] = g2
    g2b_ref[...] = g2.astype(jnp.bfloat16)


_tc2 = pl.pallas_call(
    _tc2_body,
    out_shape=(
        jax.ShapeDtypeStruct((PR, 128), jnp.float32),
        jax.ShapeDtypeStruct((PR, 128), jnp.bfloat16),
    ),
)


def _tc3_body(s2_ref, g2_ref, dinvp_ref, b2t_ref, batch_ref, wfc_ref,
              bfc_ref, out_ref):
    ssum = (s2_ref[0, :PR, :] + s2_ref[1, :PR, :]).astype(jnp.float32) \
        + g2_ref[...]
    op = jnp.maximum(dinvp_ref[...] * ssum + b2t_ref[...], 0.0)
    giota = lax.broadcasted_iota(jnp.int32, (1, G), 1)
    ones = jnp.ones((PR, 1), jnp.float32)
    sums = jnp.zeros((G, H), jnp.float32)
    cnts = jnp.zeros((G, 1), jnp.float32)
    for a in range(8):
        m = (batch_ref[:, a:a + 1] == giota).astype(jnp.float32)
        oa = op[:, a * H:(a + 1) * H]
        sums = sums + lax.dot_general(m, oa, (((0,), (0,)), ((), ())),
                                      preferred_element_type=jnp.float32)
        cnts = cnts + lax.dot_general(m, ones, (((0,), (0,)), ((), ())),
                                      preferred_element_type=jnp.float32)
    pooled = sums / jnp.maximum(cnts, 1.0)
    out_ref[...] = jnp.dot(pooled, wfc_ref[...],
                           preferred_element_type=jnp.float32) + bfc_ref[...]


_tc3 = pl.pallas_call(
    _tc3_body,
    out_shape=jax.ShapeDtypeStruct((G, NCLS), jnp.float32),
)


@jax.jit
def kernel(x, edge_index, batch, W1, b1, W2, b2, Wfc, bfc):
    ei3 = edge_index.reshape(2, ROWS, LW)
    eye8 = jnp.eye(8, dtype=jnp.float32)
    w1bd = jnp.kron(eye8, W1)
    w2bd = jnp.kron(eye8, W2)
    x8 = x.reshape(PR, 8 * DF)
    b1t = jnp.tile(b1.reshape(1, H), (1, 8))
    b2t = jnp.tile(b2.reshape(1, H), (1, 8))

    degp = _deg_kernel(ei3)
    dinv80 = _tcdinv(degp.reshape(2, DPAD // 128, 128))
    dinvp = jnp.repeat(dinv80.reshape(DPAD)[:N].reshape(PR, 8), H, axis=1)

    h1 = _tcmm(x8, w1bd)
    g1, g1b = _tcscale(h1, dinvp)
    S1 = _conv_kernel(g1b.reshape(N, H), ei3)
    g2, g2b = _tc2(S1.reshape(NC, SR, 128), g1, dinvp, b1t, w2bd)
    S2 = _conv_kernel(g2b.reshape(N, H), ei3)
    out = _tc3(S2.reshape(NC, SR, 128), g2, dinvp, b2t, batch.reshape(PR, 8),
               Wfc, bfc.reshape(1, NCLS))
    return out
